# Initial kernel scaffold; baseline (speedup 1.0000x reference)
import jax
import jax.numpy as jnp
from jax.experimental import pallas as pl


def kernel(vertex_feat, edges_feat, edges_idx, global_feat, W_e, b_e, W_v, b_v, W_g, b_g):
    raise NotImplementedError("write your pallas kernel here")


# speedup vs baseline: 8.1417x; 8.1417x over previous
import functools

import jax
import jax.numpy as jnp
from jax import lax
from jax.experimental import pallas as pl
from jax.experimental.pallas import tpu as pltpu
from jax.experimental.pallas import tpu_sc as plsc

_NC = 2
_NS = 16
_NW = _NC * _NS
_L = 16
_C = 128


def _make_sc_edge(epad, nacc, de):
    chw = epad // _NW
    ch = chw // _C
    rps = nacc // _NS
    mesh = plsc.VectorSubcoreMesh(core_axis_name="c", subcore_axis_name="s")

    def body(ps_hbm, pr_hbm, eb_hbm, sidx_hbm, ridx_hbm, newe_hbm, agg_hbm,
             sv, rv, bv, psv, prv, zv, acc, sem1, sem2):
        c = lax.axis_index("c")
        s = lax.axis_index("s")
        wid = c * _NS + s

        @pl.loop(0, rps)
        def _zero(i):
            zv[pl.ds(i, 1), pl.ds(0, _L)] = jnp.zeros((1, _L), jnp.float32)

        pltpu.sync_copy(zv, acc.at[pl.ds(pl.multiple_of(s * rps, 8), rps)])
        plsc.subcore_barrier()

        base0 = wid * chw

        @pl.loop(0, ch)
        def _chunk(j):
            base = pl.multiple_of(base0 + j * _C, _C)
            pltpu.sync_copy(sidx_hbm.at[pl.ds(base, _C)], sv)
            pltpu.sync_copy(ridx_hbm.at[pl.ds(base, _C)], rv)
            pltpu.sync_copy(eb_hbm.at[pl.ds(base, _C)], bv)
            g1 = pltpu.async_copy(ps_hbm.at[sv], psv, sem1)
            g2 = pltpu.async_copy(pr_hbm.at[rv], prv, sem2)
            g1.wait()
            g2.wait()

            @pl.loop(0, _C)
            def _row(i):
                r = (pl.ds(i, 1), pl.ds(0, _L))
                bv[r] = jnp.maximum(bv[r] + psv[r] + prv[r], 0.0)

            pltpu.sync_copy(bv, newe_hbm.at[pl.ds(base, _C)])
            pltpu.sync_copy(bv, acc.at[rv], add=True)

        plsc.subcore_barrier()
        pltpu.sync_copy(acc.at[pl.ds(pl.multiple_of(s * rps, 8), rps)],
                        agg_hbm.at[pl.ds(pl.multiple_of((c * _NS + s) * rps, 8),
                                         rps)])

    return pl.kernel(
        body,
        out_type=[
            jax.ShapeDtypeStruct((epad, de), jnp.float32),
            jax.ShapeDtypeStruct((_NC * nacc, de), jnp.float32),
        ],
        mesh=mesh,
        scratch_types=[
            pltpu.VMEM((_C,), jnp.int32),
            pltpu.VMEM((_C,), jnp.int32),
            pltpu.VMEM((_C, de), jnp.float32),
            pltpu.VMEM((_C, de), jnp.float32),
            pltpu.VMEM((_C, de), jnp.float32),
            pltpu.VMEM((rps, de), jnp.float32),
            pltpu.VMEM_SHARED((nacc, de), jnp.float32),
            pltpu.SemaphoreType.DMA,
            pltpu.SemaphoreType.DMA,
        ],
        compiler_params=pltpu.CompilerParams(use_tc_tiling_on_sc=False),
    )


def _proj_body(v_ref, w_ref, ps_ref, pr_ref):
    p = jnp.dot(v_ref[...], w_ref[...], preferred_element_type=jnp.float32)
    ps_ref[...] = p[:, :_L]
    pr_ref[...] = p[:, _L:]


def _nodebase_body(v_ref, w_ref, o_ref):
    o_ref[...] = jnp.dot(v_ref[...], w_ref[...],
                         preferred_element_type=jnp.float32)


def _edgebase_body(ef_ref, wee_ref, g_ref, weg_ref, be_ref, o_ref):
    ge = jnp.dot(g_ref[...], weg_ref[...],
                 preferred_element_type=jnp.float32) + be_ref[...]
    o_ref[...] = jnp.dot(ef_ref[...], wee_ref[...],
                         preferred_element_type=jnp.float32) + ge


def _node_body(nb_ref, agg_ref, wve_ref, g_ref, wvg_ref, bv_ref,
               newv_ref, vsum_ref, esum_ref):
    k = pl.program_id(0)
    a3 = agg_ref[...]
    a = a3[0] + a3[1]
    gv = jnp.dot(g_ref[...], wvg_ref[...],
                 preferred_element_type=jnp.float32) + bv_ref[...]
    y = jnp.maximum(
        nb_ref[...] + jnp.dot(a, wve_ref[...],
                              preferred_element_type=jnp.float32) + gv, 0.0)
    newv_ref[...] = y

    @pl.when(k == 0)
    def _():
        vsum_ref[...] = jnp.zeros_like(vsum_ref)
        esum_ref[...] = jnp.zeros_like(esum_ref)

    vsum_ref[...] += jnp.sum(y, axis=0, keepdims=True)
    esum_ref[...] += jnp.sum(a, axis=0, keepdims=True)


def _make_global_body(dg, dv, n_nodes, n_edges):
    def body(g_ref, vsum_ref, esum_ref, wg_ref, bg_ref, o_ref):
        w = wg_ref[...]
        n2g = vsum_ref[...] * (1.0 / n_nodes)
        e2g = esum_ref[...] * (1.0 / n_edges)
        x = (jnp.dot(g_ref[...], w[:dg], preferred_element_type=jnp.float32)
             + jnp.dot(n2g, w[dg:dg + dv], preferred_element_type=jnp.float32)
             + jnp.dot(e2g, w[dg + dv:], preferred_element_type=jnp.float32)
             + bg_ref[...])
        o_ref[...] = jnp.maximum(x, 0.0)
    return body


def kernel(vertex_feat, edges_feat, edges_idx, global_feat,
           W_e, b_e, W_v, b_v, W_g, b_g):
    bsz, n, dv = vertex_feat.shape
    _, e, de = edges_feat.shape
    dg = global_feat.shape[-1]
    assert bsz == 1 and de == _L

    v = vertex_feat[0]
    ef = edges_feat[0]
    s_idx = edges_idx[0, :, 0]
    r_idx = edges_idx[0, :, 1]
    g = global_feat

    w_ee = W_e[:de]
    w_sr = W_e[de:de + 2 * dv]
    w_es = W_e[de:de + dv]
    w_er = W_e[de + dv:de + 2 * dv]
    w_eg = W_e[de + 2 * dv:]
    w_vv = W_v[:dv]
    w_ve = W_v[dv:dv + de]
    w_vg = W_v[dv + de:]
    del w_sr

    w_cat = jnp.concatenate([w_es, w_er], axis=1)
    be2 = b_e[None, :]
    bv2 = b_v[None, :]
    bg2 = b_g[None, :]

    blkn = 1000
    ps, pr = pl.pallas_call(
        _proj_body,
        grid=(n // blkn,),
        in_specs=[pl.BlockSpec((blkn, dv), lambda k: (k, 0)),
                  pl.BlockSpec((dv, 2 * _L), lambda k: (0, 0))],
        out_specs=[pl.BlockSpec((blkn, _L), lambda k: (k, 0)),
                   pl.BlockSpec((blkn, _L), lambda k: (k, 0))],
        out_shape=[jax.ShapeDtypeStruct((n, _L), jnp.float32),
                   jax.ShapeDtypeStruct((n, _L), jnp.float32)],
    )(v, w_cat)

    node_base = pl.pallas_call(
        _nodebase_body,
        grid=(n // blkn,),
        in_specs=[pl.BlockSpec((blkn, dv), lambda k: (k, 0)),
                  pl.BlockSpec((dv, dv), lambda k: (0, 0))],
        out_specs=pl.BlockSpec((blkn, dv), lambda k: (k, 0)),
        out_shape=jax.ShapeDtypeStruct((n, dv), jnp.float32),
    )(v, w_vv)

    eblk = 4000
    eb = pl.pallas_call(
        _edgebase_body,
        grid=(e // eblk,),
        in_specs=[pl.BlockSpec((eblk, de), lambda k: (k, 0)),
                  pl.BlockSpec((de, de), lambda k: (0, 0)),
                  pl.BlockSpec((1, dg), lambda k: (0, 0)),
                  pl.BlockSpec((dg, de), lambda k: (0, 0)),
                  pl.BlockSpec((1, de), lambda k: (0, 0))],
        out_specs=pl.BlockSpec((eblk, de), lambda k: (k, 0)),
        out_shape=jax.ShapeDtypeStruct((e, de), jnp.float32),
    )(ef, w_ee, g, w_eg, be2)

    epad = ((e + _NW * _C - 1) // (_NW * _C)) * _NW * _C
    rps = ((n + 1 + _NS - 1) // _NS + 7) // 8 * 8
    nacc = rps * _NS
    sp = jnp.pad(s_idx, (0, epad - e), constant_values=n)
    rp = jnp.pad(r_idx, (0, epad - e), constant_values=n)
    ebp = jnp.pad(eb, ((0, epad - e), (0, 0)))
    psp = jnp.pad(ps, ((0, nacc - n), (0, 0)))
    prp = jnp.pad(pr, ((0, nacc - n), (0, 0)))

    newe_pad, agg_flat = _make_sc_edge(epad, nacc, de)(psp, prp, ebp, sp, rp)
    agg = agg_flat.reshape(_NC, nacc, de)

    newv, vsum, esum = pl.pallas_call(
        _node_body,
        grid=(n // blkn,),
        in_specs=[pl.BlockSpec((blkn, dv), lambda k: (k, 0)),
                  pl.BlockSpec((_NC, blkn, de), lambda k: (0, k, 0)),
                  pl.BlockSpec((de, dv), lambda k: (0, 0)),
                  pl.BlockSpec((1, dg), lambda k: (0, 0)),
                  pl.BlockSpec((dg, dv), lambda k: (0, 0)),
                  pl.BlockSpec((1, dv), lambda k: (0, 0))],
        out_specs=[pl.BlockSpec((blkn, dv), lambda k: (k, 0)),
                   pl.BlockSpec((1, dv), lambda k: (0, 0)),
                   pl.BlockSpec((1, de), lambda k: (0, 0))],
        out_shape=[jax.ShapeDtypeStruct((n, dv), jnp.float32),
                   jax.ShapeDtypeStruct((1, dv), jnp.float32),
                   jax.ShapeDtypeStruct((1, de), jnp.float32)],
    )(node_base, agg, w_ve, g, w_vg, bv2)

    newg = pl.pallas_call(
        _make_global_body(dg, dv, n, e),
        out_shape=jax.ShapeDtypeStruct((1, dg), jnp.float32),
    )(g, vsum, esum, W_g, bg2)

    new_e = newe_pad[:e][None]
    new_v = newv[None]
    return (new_v, new_e, newg)


# speedup vs baseline: 13.3836x; 1.6438x over previous
import functools

import jax
import jax.numpy as jnp
from jax import lax
from jax.experimental import pallas as pl
from jax.experimental.pallas import tpu as pltpu
from jax.experimental.pallas import tpu_sc as plsc

_NC = 2
_NS = 16
_NW = _NC * _NS
_L = 16
_C = 128
_SU = 4


def _make_sc_edge(e, nacc, de):
    nch = e // _C
    base_cnt = nch // _NW
    extra = nch - base_cnt * _NW
    nsup = (base_cnt + _SU) // _SU
    rps = nacc // _NS
    mesh = plsc.VectorSubcoreMesh(core_axis_name="c", subcore_axis_name="s")

    def body(ps_hbm, pr_hbm, eb_hbm, sidx_hbm, ridx_hbm, newe_hbm, agg_hbm,
             sv, rv, bv, psv, prv, zv, acc,
             sem_i, sem_b, sem_g, sem_w):
        c = lax.axis_index("c")
        s = lax.axis_index("s")
        wid = c * _NS + s

        @pl.loop(0, rps, unroll=8)
        def _zero(i):
            zv[pl.ds(i, 1), pl.ds(0, _L)] = jnp.zeros((1, _L), jnp.float32)

        pltpu.sync_copy(zv, acc.at[pl.ds(pl.multiple_of(s * rps, 8), rps)])
        plsc.subcore_barrier()

        my_cnt = jnp.where(wid < extra, base_cnt + 1, base_cnt)
        start = wid * base_cnt + jnp.minimum(wid, extra)

        def idx_cp(k, t):
            eb_off = pl.multiple_of(t * _C, _C)
            return (pltpu.make_async_copy(
                        sidx_hbm.at[pl.ds(eb_off, _C)], sv.at[k], sem_i),
                    pltpu.make_async_copy(
                        ridx_hbm.at[pl.ds(eb_off, _C)], rv.at[k], sem_i))

        def base_cp(k, t):
            eb_off = pl.multiple_of(t * _C, _C)
            return pltpu.make_async_copy(
                eb_hbm.at[pl.ds(eb_off, _C)],
                bv.at[pl.ds(k * _C, _C)], sem_b)

        def gather_cp(k):
            return (pltpu.make_async_copy(
                        ps_hbm.at[sv.at[k]],
                        psv.at[pl.ds(k * _C, _C)], sem_g),
                    pltpu.make_async_copy(
                        pr_hbm.at[rv.at[k]],
                        prv.at[pl.ds(k * _C, _C)], sem_g))

        def newe_cp(k, t):
            eb_off = pl.multiple_of(t * _C, _C)
            return pltpu.make_async_copy(
                bv.at[pl.ds(k * _C, _C)],
                newe_hbm.at[pl.ds(eb_off, _C)], sem_w)

        @pl.loop(0, nsup)
        def _super(j):
            j0 = j * _SU

            for k in range(_SU):
                @pl.when(j0 + k < my_cnt)
                def _(k=k):
                    t = start + j0 + k
                    a, b = idx_cp(k, t)
                    a.start()
                    b.start()
                    base_cp(k, t).start()

            for k in range(_SU):
                @pl.when(j0 + k < my_cnt)
                def _(k=k):
                    t = start + j0 + k
                    a, b = idx_cp(k, t)
                    a.wait()
                    b.wait()

            for k in range(_SU):
                @pl.when(j0 + k < my_cnt)
                def _(k=k):
                    a, b = gather_cp(k)
                    a.start()
                    b.start()

            for k in range(_SU):
                @pl.when(j0 + k < my_cnt)
                def _(k=k):
                    t = start + j0 + k
                    a, b = gather_cp(k)
                    a.wait()
                    b.wait()
                    base_cp(k, t).wait()

                    @pl.loop(0, _C, unroll=8)
                    def _row(i):
                        r = (pl.ds(k * _C + i, 1), pl.ds(0, _L))
                        bv[r] = jnp.maximum(bv[r] + psv[r] + prv[r], 0.0)

                    newe_cp(k, t).start()
                    pltpu.sync_copy(bv.at[pl.ds(k * _C, _C)],
                                    acc.at[rv.at[k]], add=True)
                    newe_cp(k, t).wait()

        plsc.subcore_barrier()
        pltpu.sync_copy(acc.at[pl.ds(pl.multiple_of(s * rps, 8), rps)],
                        agg_hbm.at[pl.ds(pl.multiple_of((c * _NS + s) * rps, 8),
                                         rps)])

    return pl.kernel(
        body,
        out_type=[
            jax.ShapeDtypeStruct((e, de), jnp.float32),
            jax.ShapeDtypeStruct((_NC * nacc, de), jnp.float32),
        ],
        mesh=mesh,
        scratch_types=[
            pltpu.VMEM((_SU, _C), jnp.int32),
            pltpu.VMEM((_SU, _C), jnp.int32),
            pltpu.VMEM((_SU * _C, de), jnp.float32),
            pltpu.VMEM((_SU * _C, de), jnp.float32),
            pltpu.VMEM((_SU * _C, de), jnp.float32),
            pltpu.VMEM((rps, de), jnp.float32),
            pltpu.VMEM_SHARED((nacc, de), jnp.float32),
            pltpu.SemaphoreType.DMA,
            pltpu.SemaphoreType.DMA,
            pltpu.SemaphoreType.DMA,
            pltpu.SemaphoreType.DMA,
        ],
        compiler_params=pltpu.CompilerParams(use_tc_tiling_on_sc=False),
    )


def _vproj_body(v_ref, wcat_ref, wvv_ref, ps_ref, pr_ref, nb_ref):
    x = v_ref[...]
    p = jnp.dot(x, wcat_ref[...], preferred_element_type=jnp.float32)
    ps_ref[...] = p[:, :_L]
    pr_ref[...] = p[:, _L:]
    nb_ref[...] = jnp.dot(x, wvv_ref[...], preferred_element_type=jnp.float32)


def _edgebase_body(ef_ref, wee_ref, g_ref, weg_ref, be_ref, o_ref):
    ge = jnp.dot(g_ref[...], weg_ref[...],
                 preferred_element_type=jnp.float32) + be_ref[...]
    o_ref[...] = jnp.dot(ef_ref[...], wee_ref[...],
                         preferred_element_type=jnp.float32) + ge


def _make_node_global_body(dg, dv, de, n_nodes, n_edges):
    def body(nb_ref, agg_ref, wve_ref, g_ref, wvg_ref, bv_ref,
             wg_ref, bg_ref, newv_ref, newg_ref, vsum_ref, esum_ref):
        k = pl.program_id(0)
        nsteps = pl.num_programs(0)
        a3 = agg_ref[...]
        a = a3[0] + a3[1]
        gv = jnp.dot(g_ref[...], wvg_ref[...],
                     preferred_element_type=jnp.float32) + bv_ref[...]
        y = jnp.maximum(
            nb_ref[...] + jnp.dot(a, wve_ref[...],
                                  preferred_element_type=jnp.float32) + gv,
            0.0)
        newv_ref[...] = y

        @pl.when(k == 0)
        def _():
            vsum_ref[...] = jnp.zeros_like(vsum_ref)
            esum_ref[...] = jnp.zeros_like(esum_ref)

        vsum_ref[...] += jnp.sum(y, axis=0, keepdims=True)
        esum_ref[...] += jnp.sum(a, axis=0, keepdims=True)

        @pl.when(k == nsteps - 1)
        def _():
            w = wg_ref[...]
            n2g = vsum_ref[...] * (1.0 / n_nodes)
            e2g = esum_ref[...] * (1.0 / n_edges)
            x = (jnp.dot(g_ref[...], w[:dg],
                         preferred_element_type=jnp.float32)
                 + jnp.dot(n2g, w[dg:dg + dv],
                           preferred_element_type=jnp.float32)
                 + jnp.dot(e2g, w[dg + dv:],
                           preferred_element_type=jnp.float32)
                 + bg_ref[...])
            newg_ref[...] = jnp.maximum(x, 0.0)
    return body


def kernel(vertex_feat, edges_feat, edges_idx, global_feat,
           W_e, b_e, W_v, b_v, W_g, b_g):
    bsz, n, dv = vertex_feat.shape
    _, e, de = edges_feat.shape
    dg = global_feat.shape[-1]
    assert bsz == 1 and de == _L and e % _C == 0

    v = vertex_feat[0]
    ef = edges_feat[0]
    s_idx = edges_idx[0, :, 0]
    r_idx = edges_idx[0, :, 1]
    g = global_feat

    w_ee = W_e[:de]
    w_es = W_e[de:de + dv]
    w_er = W_e[de + dv:de + 2 * dv]
    w_eg = W_e[de + 2 * dv:]
    w_vv = W_v[:dv]
    w_ve = W_v[dv:dv + de]
    w_vg = W_v[dv + de:]

    w_cat = jnp.concatenate([w_es, w_er], axis=1)
    be2 = b_e[None, :]
    bv2 = b_v[None, :]
    bg2 = b_g[None, :]

    blkn = 1000
    ps, pr, node_base = pl.pallas_call(
        _vproj_body,
        grid=(n // blkn,),
        in_specs=[pl.BlockSpec((blkn, dv), lambda k: (k, 0)),
                  pl.BlockSpec((dv, 2 * _L), lambda k: (0, 0)),
                  pl.BlockSpec((dv, dv), lambda k: (0, 0))],
        out_specs=[pl.BlockSpec((blkn, _L), lambda k: (k, 0)),
                   pl.BlockSpec((blkn, _L), lambda k: (k, 0)),
                   pl.BlockSpec((blkn, dv), lambda k: (k, 0))],
        out_shape=[jax.ShapeDtypeStruct((n, _L), jnp.float32),
                   jax.ShapeDtypeStruct((n, _L), jnp.float32),
                   jax.ShapeDtypeStruct((n, dv), jnp.float32)],
    )(v, w_cat, w_vv)

    eblk = 4000
    eb = pl.pallas_call(
        _edgebase_body,
        grid=(e // eblk,),
        in_specs=[pl.BlockSpec((eblk, de), lambda k: (k, 0)),
                  pl.BlockSpec((de, de), lambda k: (0, 0)),
                  pl.BlockSpec((1, dg), lambda k: (0, 0)),
                  pl.BlockSpec((dg, de), lambda k: (0, 0)),
                  pl.BlockSpec((1, de), lambda k: (0, 0))],
        out_specs=pl.BlockSpec((eblk, de), lambda k: (k, 0)),
        out_shape=jax.ShapeDtypeStruct((e, de), jnp.float32),
    )(ef, w_ee, g, w_eg, be2)

    rps = ((n + _NS - 1) // _NS + 7) // 8 * 8
    nacc = rps * _NS

    newe, agg_flat = _make_sc_edge(e, nacc, de)(ps, pr, eb, s_idx, r_idx)
    agg = agg_flat.reshape(_NC, nacc, de)

    newv, newg, _, _ = pl.pallas_call(
        _make_node_global_body(dg, dv, de, n, e),
        grid=(n // blkn,),
        in_specs=[pl.BlockSpec((blkn, dv), lambda k: (k, 0)),
                  pl.BlockSpec((_NC, blkn, de), lambda k: (0, k, 0)),
                  pl.BlockSpec((de, dv), lambda k: (0, 0)),
                  pl.BlockSpec((1, dg), lambda k: (0, 0)),
                  pl.BlockSpec((dg, dv), lambda k: (0, 0)),
                  pl.BlockSpec((1, dv), lambda k: (0, 0)),
                  pl.BlockSpec((dg + dv + de, dg), lambda k: (0, 0)),
                  pl.BlockSpec((1, dg), lambda k: (0, 0))],
        out_specs=[pl.BlockSpec((blkn, dv), lambda k: (k, 0)),
                   pl.BlockSpec((1, dg), lambda k: (0, 0)),
                   pl.BlockSpec((1, dv), lambda k: (0, 0)),
                   pl.BlockSpec((1, de), lambda k: (0, 0))],
        out_shape=[jax.ShapeDtypeStruct((n, dv), jnp.float32),
                   jax.ShapeDtypeStruct((1, dg), jnp.float32),
                   jax.ShapeDtypeStruct((1, dv), jnp.float32),
                   jax.ShapeDtypeStruct((1, de), jnp.float32)],
    )(node_base, agg, w_ve, g, w_vg, bv2, W_g, bg2)

    return (newv[None], newe[None], newg)
